# Initial kernel scaffold; baseline (speedup 1.0000x reference)
import jax
import jax.numpy as jnp
from jax.experimental import pallas as pl


def kernel(obs_emb, adj, observed_mask, observed_tp, tp_emb_tensor, lengths, avg_interval, var_prior_emb_tensor, Wu, bu, Wr, br, Wc, bc, pw1, pb1, pw2, pb2):
    raise NotImplementedError("write your pallas kernel here")


# speedup vs baseline: 81.7568x; 81.7568x over previous
import jax
import jax.numpy as jnp
from jax.experimental import pallas as pl
from jax.experimental.pallas import tpu as pltpu

BATCH = 8
STEPS = 512
NUM_NODES = 64
D_MODEL = 64
ROWS = BATCH * NUM_NODES
VE = 5
RARITY_ALPHA = 0.5
CHUNK = 16


def _prep_kernel(vpe_ref, pw1_ref, pb1_ref, pw2_ref, pb2_ref, mask_ref,
                 wrar_ref, bstack_ref, vv_ref, vt_ref, wrar_eff_ref, beff_ref):
    vpe = vpe_ref[...]
    hmid = jax.nn.relu(
        jnp.dot(vpe, pw1_ref[...], preferred_element_type=jnp.float32)
        + pb1_ref[...])
    vv = jnp.dot(hmid, pw2_ref[...], preferred_element_type=jnp.float32) \
        + pb2_ref[...]
    vv_ref[...] = vv
    vt_ref[...] = jnp.sum(mask_ref[...], axis=1)
    wrar_eff_ref[...] = jnp.dot(vv, wrar_ref[...],
                                preferred_element_type=jnp.float32)
    beff_ref[...] = jnp.dot(vv, bstack_ref[...],
                            preferred_element_type=jnp.float32)


def _main_kernel(obs_ref, mask_ref, avg_ref, wx_ref, whru_ref, whc_ref,
                 vvt_ref, wrar_t_ref, beff_t_ref, vt_ref, len_ref,
                 out_ref, h_scr, out_scr):
    k = pl.program_id(0)

    @pl.when(k == 0)
    def _init():
        h_scr[...] = jnp.zeros_like(h_scr)
        out_scr[...] = jnp.zeros_like(out_scr)

    wx = wx_ref[...]
    whru = whru_ref[...]
    whc = whc_ref[...]
    vvt = vvt_ref[...]
    wrar_t = wrar_t_ref[...]
    beff_t = beff_t_ref[...]
    vt = vt_ref[...]
    lens = len_ref[...]

    def outer(vec):
        return jnp.concatenate([vvt[d:d + 1] * vec for d in range(VE)], axis=0)

    def body(i, carry):
        h, out = carry
        obs_t = obs_ref[i]
        m = mask_ref[i] > 0.0
        rar = RARITY_ALPHA * jnp.tanh(avg_ref[i] / (vt + 1.0))

        zx = outer(obs_t)
        xpre = jnp.dot(wx, zx, preferred_element_type=jnp.float32) \
            + rar * wrar_t + beff_t

        zh = outer(h)
        pre_ru = xpre[0:128] + jnp.dot(whru, zh,
                                       preferred_element_type=jnp.float32)
        r = jax.nn.sigmoid(pre_ru[0:64])
        u = jax.nn.sigmoid(pre_ru[64:128])
        h_reset = jnp.where(m, r * h, h)

        zc = outer(h_reset)
        cand = jnp.tanh(xpre[128:192] + jnp.dot(
            whc, zc, preferred_element_type=jnp.float32))
        h_new = jnp.where(m, (1.0 - u) * h_reset + u * cand, h_reset)

        t = k * CHUNK + i
        sel = lens == (t + 1)
        out_new = jnp.where(sel, h_new, out)
        return h_new, out_new

    h0 = h_scr[...]
    o0 = out_scr[...]
    h, out = jax.lax.fori_loop(0, CHUNK, body, (h0, o0))
    h_scr[...] = h
    out_scr[...] = out
    out_ref[...] = out


def _run(obs_T, mask_T, avg_T, wx_T, whru_T, whc_T, vv_T, wrar_T, beff_T,
         vt_T, len_row):
    grid = STEPS // CHUNK
    full = lambda shape: pl.BlockSpec(shape, lambda k: (0,) * len(shape))
    return pl.pallas_call(
        _main_kernel,
        grid=(grid,),
        in_specs=[
            pl.BlockSpec((CHUNK, D_MODEL, ROWS), lambda k: (k, 0, 0)),
            pl.BlockSpec((CHUNK, 1, ROWS), lambda k: (k, 0, 0)),
            pl.BlockSpec((CHUNK, 1, ROWS), lambda k: (k, 0, 0)),
            full((192, 320)),
            full((128, 320)),
            full((64, 320)),
            full((VE, ROWS)),
            full((192, ROWS)),
            full((192, ROWS)),
            full((1, ROWS)),
            full((1, ROWS)),
        ],
        out_specs=pl.BlockSpec((D_MODEL, ROWS), lambda k: (0, 0)),
        out_shape=jax.ShapeDtypeStruct((D_MODEL, ROWS), jnp.float32),
        scratch_shapes=[
            pltpu.VMEM((D_MODEL, ROWS), jnp.float32),
            pltpu.VMEM((D_MODEL, ROWS), jnp.float32),
        ],
    )(obs_T, mask_T, avg_T, wx_T, whru_T, whc_T, vv_T, wrar_T, beff_T,
      vt_T, len_row)


def kernel(obs_emb, adj, observed_mask, observed_tp, tp_emb_tensor, lengths,
           avg_interval, var_prior_emb_tensor, Wu, bu, Wr, br, Wc, bc,
           pw1, pb1, pw2, pb2):
    del adj, observed_tp, tp_emb_tensor

    def stack_x(W):
        return W[:, :D_MODEL, :].reshape(VE * D_MODEL, D_MODEL)

    def stack_h(W):
        return W[:, D_MODEL + 1:, :].reshape(VE * D_MODEL, D_MODEL)

    wx = jnp.concatenate([stack_x(Wr), stack_x(Wu), stack_x(Wc)], axis=1)
    wh = jnp.concatenate([stack_h(Wr), stack_h(Wu), stack_h(Wc)], axis=1)
    wx_T = wx.T
    whru_T = wh[:, :128].T
    whc_T = wh[:, 128:].T
    wrar = jnp.concatenate(
        [Wr[:, D_MODEL, :], Wu[:, D_MODEL, :], Wc[:, D_MODEL, :]], axis=1)
    bstack = jnp.concatenate([br, bu, bc], axis=1)

    vv, vt, wrar_eff, beff = pl.pallas_call(
        _prep_kernel,
        out_shape=(
            jax.ShapeDtypeStruct((NUM_NODES, VE), jnp.float32),
            jax.ShapeDtypeStruct((BATCH, NUM_NODES), jnp.float32),
            jax.ShapeDtypeStruct((NUM_NODES, 192), jnp.float32),
            jax.ShapeDtypeStruct((NUM_NODES, 192), jnp.float32),
        ),
    )(var_prior_emb_tensor, pw1, pb1.reshape(1, -1), pw2, pb2.reshape(1, -1),
      observed_mask, wrar, bstack)

    def bcast_rows(a):
        return jnp.broadcast_to(a.T[:, None, :], (a.shape[1], BATCH,
                                                  NUM_NODES)).reshape(
                                                      a.shape[1], ROWS)

    vv_T = bcast_rows(vv)
    wrar_T = bcast_rows(wrar_eff)
    beff_T = bcast_rows(beff)
    vt_T = vt.reshape(1, ROWS)
    len_row = jnp.broadcast_to(
        lengths.reshape(BATCH, 1).astype(jnp.int32), (BATCH, NUM_NODES)
    ).reshape(1, ROWS)

    obs_T = obs_emb.transpose(1, 3, 0, 2).reshape(STEPS, D_MODEL, ROWS)
    mask_T = observed_mask.transpose(1, 0, 2).reshape(STEPS, 1, ROWS)
    avg_T = avg_interval.transpose(1, 0, 2).reshape(STEPS, 1, ROWS)

    out_T = _run(obs_T, mask_T, avg_T, wx_T, whru_T, whc_T, vv_T, wrar_T,
                 beff_T, vt_T, len_row)
    return out_T.reshape(D_MODEL, BATCH, NUM_NODES).transpose(1, 2, 0)
